# Initial kernel scaffold; baseline (speedup 1.0000x reference)
import jax
import jax.numpy as jnp
from jax.experimental import pallas as pl


def kernel(z, x, edge_index):
    raise NotImplementedError("write your pallas kernel here")


# speedup vs baseline: 3.5271x; 3.5271x over previous
import functools

import jax
import jax.numpy as jnp
from jax import lax
from jax.experimental import pallas as pl
from jax.experimental.pallas import tpu as pltpu
from jax.experimental.pallas import tpu_sc as plsc

N_NODES = 10000
N_EDGES = 160000
D_FEAT = 256

L = 16
NC = 2
NS = 16
NW = NC * NS
EPW = 5120
E_PAD = NW * EPW
CH = 64
NCHUNK = EPW // CH
GPC = CH // L
N_PAD = 10240
NPT = N_PAD // NS

_mesh = plsc.VectorSubcoreMesh(
    core_axis_name="c", subcore_axis_name="s", num_cores=NC, num_subcores=NS
)


@functools.partial(
    pl.kernel,
    out_type=[
        jax.ShapeDtypeStruct((E_PAD,), jnp.float32),
        jax.ShapeDtypeStruct((NC, N_PAD), jnp.float32),
    ],
    mesh=_mesh,
    compiler_params=pltpu.CompilerParams(needs_layout_passes=False),
    scratch_types=[
        pltpu.VMEM((EPW,), jnp.int32),
        pltpu.VMEM((EPW,), jnp.int32),
        pltpu.VMEM((CH, D_FEAT), jnp.float32),
        pltpu.VMEM((CH, D_FEAT), jnp.float32),
        pltpu.VMEM((L * L,), jnp.float32),
        pltpu.VMEM((L,), jnp.int32),
        pltpu.VMEM((L,), jnp.float32),
        pltpu.VMEM((EPW,), jnp.float32),
        pltpu.VMEM((N_PAD,), jnp.float32),
        pltpu.VMEM((NS, NPT), jnp.float32),
        pltpu.VMEM((NPT,), jnp.float32),
        pltpu.VMEM_SHARED((NS, N_PAD), jnp.float32),
        pltpu.SemaphoreType.DMA,
        pltpu.SemaphoreType.DMA,
    ],
)
def _edge_scores(src_hbm, dst_hbm, z_hbm, x_hbm, m_hbm, part_hbm,
                 src_v, dst_v, zrows, xrows, pmat, sbuf, cbuf, m_v, acc,
                 redbuf, red_v, shared, sem_z, sem_x):
    c = lax.axis_index("c")
    s = lax.axis_index("s")
    wid = s * NC + c
    ebase = wid * EPW

    zeros = jnp.zeros((L,), jnp.float32)

    def zero_body(i, carry):
        acc[pl.ds(i * L, L)] = zeros
        return carry

    lax.fori_loop(0, N_PAD // L, zero_body, 0)

    pltpu.sync_copy(src_hbm.at[pl.ds(ebase, EPW)], src_v)
    pltpu.sync_copy(dst_hbm.at[pl.ds(ebase, EPW)], dst_v)

    iota = lax.iota(jnp.int32, L)

    def group_body(off, g, carry):
        gbase = off + g * L
        for e in range(L):
            el = g * L + e
            p = zrows[el, pl.ds(0, L)] * xrows[el, pl.ds(0, L)]
            for k in range(1, D_FEAT // L):
                p = p + zrows[el, pl.ds(k * L, L)] * xrows[el, pl.ds(k * L, L)]
            pmat[pl.ds(e * L, L)] = p
        rowbase = iota * L
        tot = plsc.load_gather(pmat, [rowbase])
        for j in range(1, L):
            tot = tot + plsc.load_gather(pmat, [rowbase + j])
        m16 = jnp.exp(tot)
        eid = ebase + gbase + iota
        m16 = jnp.where(eid < N_EDGES, m16, 0.0)
        m_v[pl.ds(gbase, L)] = m16
        dst16 = dst_v[pl.ds(gbase, L)]
        sd, sm = plsc.sort_key_val(dst16, m16)
        sbuf[pl.ds(0, L)] = sd
        cum = sm
        for sh in (1, 2, 4, 8):
            idx = jnp.maximum(iota - sh, 0)
            cbuf[pl.ds(0, L)] = cum
            prev = plsc.load_gather(cbuf, [idx])
            prevk = plsc.load_gather(sbuf, [idx])
            seg = (prevk == sd) & (iota >= sh)
            cum = cum + jnp.where(seg, prev, 0.0)
        nxt = plsc.load_gather(sbuf, [jnp.minimum(iota + 1, L - 1)])
        is_last = (sd != nxt) | (iota == L - 1)
        plsc.addupdate_scatter(acc, [sd], cum, mask=is_last)
        return carry

    def chunk_body(ci, carry):
        off = ci * CH
        cpz = pltpu.async_copy(z_hbm.at[src_v.at[pl.ds(off, CH)]], zrows, sem_z)
        cpx = pltpu.async_copy(x_hbm.at[dst_v.at[pl.ds(off, CH)]], xrows, sem_x)
        cpz.wait()
        cpx.wait()

        def g_body(g, cc):
            return group_body(off, g, cc)

        return lax.fori_loop(0, GPC, g_body, carry)

    lax.fori_loop(0, NCHUNK, chunk_body, 0)

    pltpu.sync_copy(m_v, m_hbm.at[pl.ds(ebase, EPW)])

    pltpu.sync_copy(acc, shared.at[s])
    plsc.subcore_barrier()
    for r in range(NS):
        pltpu.sync_copy(shared.at[r, pl.ds(s * NPT, NPT)], redbuf.at[r])

    def red_body(i, carry):
        t = redbuf[0, pl.ds(i * L, L)]
        for r in range(1, NS):
            t = t + redbuf[r, pl.ds(i * L, L)]
        red_v[pl.ds(i * L, L)] = t
        return carry

    lax.fori_loop(0, NPT // L, red_body, 0)
    pltpu.sync_copy(red_v, part_hbm.at[c, pl.ds(s * NPT, NPT)])


@functools.partial(
    pl.kernel,
    out_type=jax.ShapeDtypeStruct((E_PAD,), jnp.float32),
    mesh=_mesh,
    compiler_params=pltpu.CompilerParams(needs_layout_passes=False),
    scratch_types=[
        pltpu.VMEM((N_PAD,), jnp.float32),
        pltpu.VMEM((N_PAD,), jnp.float32),
        pltpu.VMEM((EPW,), jnp.float32),
        pltpu.VMEM((EPW,), jnp.int32),
        pltpu.VMEM((EPW,), jnp.float32),
    ],
)
def _normalize(m_hbm, dst_hbm, part_hbm, out_hbm, p0, den, m_v, dst_v, prob_v):
    c = lax.axis_index("c")
    s = lax.axis_index("s")
    wid = s * NC + c
    ebase = wid * EPW

    pltpu.sync_copy(part_hbm.at[0], p0)
    pltpu.sync_copy(part_hbm.at[1], den)

    def add_body(i, carry):
        den[pl.ds(i * L, L)] = den[pl.ds(i * L, L)] + p0[pl.ds(i * L, L)]
        return carry

    lax.fori_loop(0, N_PAD // L, add_body, 0)

    pltpu.sync_copy(m_hbm.at[pl.ds(ebase, EPW)], m_v)
    pltpu.sync_copy(dst_hbm.at[pl.ds(ebase, EPW)], dst_v)

    def g_body(gi, carry):
        m16 = m_v[pl.ds(gi * L, L)]
        dst16 = dst_v[pl.ds(gi * L, L)]
        d16 = plsc.load_gather(den, [dst16])
        prob_v[pl.ds(gi * L, L)] = m16 / d16
        return carry

    lax.fori_loop(0, EPW // L, g_body, 0)
    pltpu.sync_copy(prob_v, out_hbm.at[pl.ds(ebase, EPW)])


def kernel(z, x, edge_index):
    src = edge_index[0].astype(jnp.int32)
    dst = edge_index[1].astype(jnp.int32)
    pad = E_PAD - N_EDGES
    srcp = jnp.concatenate([src, jnp.zeros((pad,), jnp.int32)])
    dstp = jnp.concatenate([dst, jnp.zeros((pad,), jnp.int32)])
    m, part = _edge_scores(srcp, dstp, z, x)
    prob = _normalize(m, dstp, part)
    return prob[:N_EDGES]


# speedup vs baseline: 4.5958x; 1.3030x over previous
import functools

import jax
import jax.numpy as jnp
from jax import lax
from jax.experimental import pallas as pl
from jax.experimental.pallas import tpu as pltpu
from jax.experimental.pallas import tpu_sc as plsc

N_NODES = 10000
N_EDGES = 160000
D_FEAT = 256

L = 16
NC = 2
NS = 16
NW = NC * NS
EPW = 5120
E_PAD = NW * EPW
CH = 64
NCHUNK = EPW // CH
GPC = CH // L
N_PAD = 10240
NPT = N_PAD // NS

_mesh = plsc.VectorSubcoreMesh(
    core_axis_name="c", subcore_axis_name="s", num_cores=NC, num_subcores=NS
)


@functools.partial(
    pl.kernel,
    out_type=[
        jax.ShapeDtypeStruct((E_PAD,), jnp.float32),
        jax.ShapeDtypeStruct((NC, N_PAD), jnp.float32),
    ],
    mesh=_mesh,
    compiler_params=pltpu.CompilerParams(needs_layout_passes=False),
    scratch_types=[
        pltpu.VMEM((EPW,), jnp.int32),
        pltpu.VMEM((EPW,), jnp.int32),
        pltpu.VMEM((CH, D_FEAT), jnp.float32),
        pltpu.VMEM((CH, D_FEAT), jnp.float32),
        pltpu.VMEM((CH, D_FEAT), jnp.float32),
        pltpu.VMEM((CH, D_FEAT), jnp.float32),
        pltpu.VMEM((L * L,), jnp.float32),
        pltpu.VMEM((L,), jnp.int32),
        pltpu.VMEM((L,), jnp.float32),
        pltpu.VMEM((EPW,), jnp.float32),
        pltpu.VMEM((N_PAD,), jnp.float32),
        pltpu.VMEM((NS, NPT), jnp.float32),
        pltpu.VMEM((NPT,), jnp.float32),
        pltpu.VMEM_SHARED((NS, N_PAD), jnp.float32),
        pltpu.SemaphoreType.DMA,
        pltpu.SemaphoreType.DMA,
        pltpu.SemaphoreType.DMA,
        pltpu.SemaphoreType.DMA,
    ],
)
def _edge_scores(src_hbm, dst_hbm, z_hbm, x_hbm, m_hbm, part_hbm,
                 src_v, dst_v, zrows0, xrows0, zrows1, xrows1,
                 pmat, sbuf, cbuf, m_v, acc,
                 redbuf, red_v, shared, sem_z0, sem_x0, sem_z1, sem_x1):
    c = lax.axis_index("c")
    s = lax.axis_index("s")
    wid = s * NC + c
    ebase = wid * EPW

    zeros = jnp.zeros((L,), jnp.float32)

    def zero_body(i, carry):
        acc[pl.ds(i * L, L)] = zeros
        return carry

    lax.fori_loop(0, N_PAD // L, zero_body, 0)

    pltpu.sync_copy(src_hbm.at[pl.ds(ebase, EPW)], src_v)
    pltpu.sync_copy(dst_hbm.at[pl.ds(ebase, EPW)], dst_v)

    iota = lax.iota(jnp.int32, L)

    def group_body(off, g, zrows, xrows, carry):
        gbase = off + g * L
        for e in range(L):
            el = g * L + e
            p = zrows[el, pl.ds(0, L)] * xrows[el, pl.ds(0, L)]
            for k in range(1, D_FEAT // L):
                p = p + zrows[el, pl.ds(k * L, L)] * xrows[el, pl.ds(k * L, L)]
            pmat[pl.ds(e * L, L)] = p
        rowbase = iota * L
        tot = plsc.load_gather(pmat, [rowbase])
        for j in range(1, L):
            tot = tot + plsc.load_gather(pmat, [rowbase + j])
        m16 = jnp.exp(tot)
        eid = ebase + gbase + iota
        m16 = jnp.where(eid < N_EDGES, m16, 0.0)
        m_v[pl.ds(gbase, L)] = m16
        dst16 = dst_v[pl.ds(gbase, L)]
        sd, sm = plsc.sort_key_val(dst16, m16)
        sbuf[pl.ds(0, L)] = sd
        cum = sm
        for sh in (1, 2, 4, 8):
            idx = jnp.maximum(iota - sh, 0)
            cbuf[pl.ds(0, L)] = cum
            prev = plsc.load_gather(cbuf, [idx])
            prevk = plsc.load_gather(sbuf, [idx])
            seg = (prevk == sd) & (iota >= sh)
            cum = cum + jnp.where(seg, prev, 0.0)
        nxt = plsc.load_gather(sbuf, [jnp.minimum(iota + 1, L - 1)])
        is_last = (sd != nxt) | (iota == L - 1)
        plsc.addupdate_scatter(acc, [sd], cum, mask=is_last)
        return carry

    bufs = ((zrows0, xrows0, sem_z0, sem_x0), (zrows1, xrows1, sem_z1, sem_x1))

    def start_gather(ci, zbuf, xbuf, semz, semx):
        off = ci * CH
        pltpu.async_copy(z_hbm.at[src_v.at[pl.ds(off, CH)]], zbuf, semz)
        pltpu.async_copy(x_hbm.at[dst_v.at[pl.ds(off, CH)]], xbuf, semx)

    def wait_gather(zbuf, xbuf, semz, semx):
        pltpu.make_async_copy(z_hbm.at[pl.ds(0, CH)], zbuf, semz).wait()
        pltpu.make_async_copy(x_hbm.at[pl.ds(0, CH)], xbuf, semx).wait()

    start_gather(0, *bufs[0])

    def pair_body(i, carry):
        for b in range(2):
            ci = 2 * i + b
            zbuf, xbuf, semz, semx = bufs[b]
            wait_gather(zbuf, xbuf, semz, semx)
            if b == 0:
                start_gather(ci + 1, *bufs[1])
            else:
                @pl.when(i < NCHUNK // 2 - 1)
                def _():
                    start_gather(ci + 1, *bufs[0])

            def g_body(g, cc):
                return group_body(ci * CH, g, zbuf, xbuf, cc)

            lax.fori_loop(0, GPC, g_body, 0)
        return carry

    lax.fori_loop(0, NCHUNK // 2, pair_body, 0)

    pltpu.sync_copy(m_v, m_hbm.at[pl.ds(ebase, EPW)])

    pltpu.sync_copy(acc, shared.at[s])
    plsc.subcore_barrier()
    for r in range(NS):
        pltpu.sync_copy(shared.at[r, pl.ds(s * NPT, NPT)], redbuf.at[r])

    def red_body(i, carry):
        t = redbuf[0, pl.ds(i * L, L)]
        for r in range(1, NS):
            t = t + redbuf[r, pl.ds(i * L, L)]
        red_v[pl.ds(i * L, L)] = t
        return carry

    lax.fori_loop(0, NPT // L, red_body, 0)
    pltpu.sync_copy(red_v, part_hbm.at[c, pl.ds(s * NPT, NPT)])


@functools.partial(
    pl.kernel,
    out_type=jax.ShapeDtypeStruct((E_PAD,), jnp.float32),
    mesh=_mesh,
    compiler_params=pltpu.CompilerParams(needs_layout_passes=False),
    scratch_types=[
        pltpu.VMEM((N_PAD,), jnp.float32),
        pltpu.VMEM((N_PAD,), jnp.float32),
        pltpu.VMEM((EPW,), jnp.float32),
        pltpu.VMEM((EPW,), jnp.int32),
        pltpu.VMEM((EPW,), jnp.float32),
    ],
)
def _normalize(m_hbm, dst_hbm, part_hbm, out_hbm, p0, den, m_v, dst_v, prob_v):
    c = lax.axis_index("c")
    s = lax.axis_index("s")
    wid = s * NC + c
    ebase = wid * EPW

    pltpu.sync_copy(part_hbm.at[0], p0)
    pltpu.sync_copy(part_hbm.at[1], den)

    def add_body(i, carry):
        den[pl.ds(i * L, L)] = den[pl.ds(i * L, L)] + p0[pl.ds(i * L, L)]
        return carry

    lax.fori_loop(0, N_PAD // L, add_body, 0)

    pltpu.sync_copy(m_hbm.at[pl.ds(ebase, EPW)], m_v)
    pltpu.sync_copy(dst_hbm.at[pl.ds(ebase, EPW)], dst_v)

    def g_body(gi, carry):
        m16 = m_v[pl.ds(gi * L, L)]
        dst16 = dst_v[pl.ds(gi * L, L)]
        d16 = plsc.load_gather(den, [dst16])
        prob_v[pl.ds(gi * L, L)] = m16 / d16
        return carry

    lax.fori_loop(0, EPW // L, g_body, 0)
    pltpu.sync_copy(prob_v, out_hbm.at[pl.ds(ebase, EPW)])


def kernel(z, x, edge_index):
    src = edge_index[0].astype(jnp.int32)
    dst = edge_index[1].astype(jnp.int32)
    pad = E_PAD - N_EDGES
    srcp = jnp.concatenate([src, jnp.zeros((pad,), jnp.int32)])
    dstp = jnp.concatenate([dst, jnp.zeros((pad,), jnp.int32)])
    m, part = _edge_scores(srcp, dstp, z, x)
    prob = _normalize(m, dstp, part)
    return prob[:N_EDGES]
